# Initial kernel scaffold; baseline (speedup 1.0000x reference)
import jax
import jax.numpy as jnp
from jax.experimental import pallas as pl


def kernel(x, edge_index, edge_attr, W_lin, W_edge, in_proj_w, in_proj_b, out_proj_w, out_proj_b):
    raise NotImplementedError("write your pallas kernel here")


# speedup vs baseline: 73.4996x; 73.4996x over previous
import functools

import jax
import jax.numpy as jnp
from jax import lax
from jax.experimental import pallas as pl
from jax.experimental.pallas import tpu as pltpu

N = 256
E = 32
D = 128
H = 8
DH = D // H
SCALE = 1.0 / (DH ** 0.5)
NEG = -jnp.inf


def _dot(a, b):
    return jnp.dot(a, b, preferred_element_type=jnp.float32,
                   precision=lax.Precision.HIGHEST)


def _attn_body(x_ref, ei_ref, ea_ref, wlin_t_ref, wedge_t_ref, wq_t_ref,
               wk_t_ref, wv_t_ref, bqkv_ref, wout_t_ref, bout_ref, out_ref):
    Mf = (ei_ref[...] != 0).astype(jnp.float32)
    Mt = Mf.T
    Wh = _dot(x_ref[...], wlin_t_ref[...])
    We = _dot(ea_ref[...], wedge_t_ref[...])
    bq = bqkv_ref[0:1, :]
    bk = bqkv_ref[1:2, :]
    bv = bqkv_ref[2:3, :]
    Q = _dot(Wh, wq_t_ref[...]) + bq
    Kx = _dot(Wh, wk_t_ref[...])
    V = _dot(Wh, wv_t_ref[...]) + bv
    Ke = _dot(We, wk_t_ref[...]) + bk

    U = _dot(Mt, Mf)
    has = jnp.sum(Mt, axis=1, keepdims=True) > 0.0

    heads = []
    for h in range(H):
        sl = slice(h * DH, (h + 1) * DH)
        Qh, Kxh, Vh, Keh = Q[:, sl], Kx[:, sl], V[:, sl], Ke[:, sl]
        Ah = _dot(Qh, Kxh.T) * SCALE
        Bh = _dot(Qh, Keh.T) * SCALE
        mB = jnp.max(jnp.where(Mt > 0, Bh, NEG), axis=1, keepdims=True)
        s = jnp.where(Mt > 0, jnp.exp(Bh - mB), 0.0)
        Ch = _dot(s, Mf)
        mA = jnp.max(jnp.where(U > 0, Ah, NEG), axis=1, keepdims=True)
        Wgt = Ch * jnp.exp(Ah - mA)
        Z = jnp.sum(Wgt, axis=1, keepdims=True)
        heads.append(_dot(Wgt, Vh) / Z)
    attn = jnp.concatenate(heads, axis=1)
    o = _dot(attn, wout_t_ref[...]) + bout_ref[...]
    o = jnp.where(has, o, 0.0)
    out_ref[...] = jnp.maximum(o, 0.0)


@jax.jit
def kernel(x, edge_index, edge_attr, W_lin, W_edge, in_proj_w, in_proj_b,
           out_proj_w, out_proj_b):
    Wq, Wk, Wv = jnp.split(in_proj_w, 3, axis=0)
    bqkv = in_proj_b.reshape(3, D)
    call = pl.pallas_call(
        _attn_body,
        out_shape=jax.ShapeDtypeStruct((N, D), jnp.float32),
    )
    return call(x, edge_index.astype(jnp.int32), edge_attr, W_lin.T, W_edge.T,
                Wq.T, Wk.T, Wv.T, bqkv, out_proj_w.T,
                out_proj_b.reshape(1, D))
